# Initial kernel scaffold; baseline (speedup 1.0000x reference)
import jax
import jax.numpy as jnp
from jax.experimental import pallas as pl


def kernel(z, past_exo_cont, past_exo_cat, tables, W_proj, b_proj, W_gate, b_gate):
    raise NotImplementedError("write your pallas kernel here")


# speedup vs baseline: 9.5108x; 9.5108x over previous
import functools

import jax
import jax.numpy as jnp
from jax import lax
from jax.experimental import pallas as pl
from jax.experimental.pallas import tpu as pltpu
from jax.experimental.pallas import tpu_sc as plsc

B = 4096
T = 50
CONT = 16
NCAT = 26
V = 100000
ED = 32
ZD = 1024
IN_DIM = CONT + NCAT * ED
VPAD = 896

NC = 2
NS = 16
NW = NC * NS
BPW = B // NW

NIDS = T * NCAT
NSTREAM = 11
SLEN = 120
IDS_PAD = NSTREAM * SLEN

UNROLL = 10


def _pool_body(ids_hbm, cont_hbm, tab_hbm, v_hbm, idx_v, rows_v, cont_v, out_v, sem):
    wid = lax.axis_index("s") * NC + lax.axis_index("c")
    base = wid * BPW

    zeros16 = jnp.zeros((16,), jnp.float32)
    for k in range(IN_DIM, VPAD, 16):
        out_v[pl.ds(k, 16)] = zeros16

    def body(i, carry):
        b = base + i
        pltpu.sync_copy(ids_hbm.at[b], idx_v)
        pltpu.sync_copy(cont_hbm.at[b], cont_v)
        copies = [
            pltpu.async_copy(
                tab_hbm.at[idx_v.at[j]],
                rows_v.at[pl.ds(j * SLEN, SLEN)],
                sem,
            )
            for j in range(NSTREAM)
        ]
        for cp in copies:
            cp.wait()

        def cont_step(t, acc):
            return acc + cont_v[t, :]

        cacc = lax.fori_loop(0, T, cont_step, zeros16)
        out_v[pl.ds(0, 16)] = cacc * (1.0 / T)

        for f in range(NCAT):
            def cat_step(t5, accs, f=f):
                a0, a1 = accs
                r0 = t5 * (UNROLL * NCAT) + f
                for u in range(UNROLL):
                    r = r0 + u * NCAT
                    a0 = a0 + rows_v[r, pl.ds(0, 16)]
                    a1 = a1 + rows_v[r, pl.ds(16, 16)]
                return a0, a1

            a0, a1 = lax.fori_loop(0, T // UNROLL, cat_step, (zeros16, zeros16))
            out_v[pl.ds(CONT + f * ED, 16)] = a0 * (1.0 / T)
            out_v[pl.ds(CONT + f * ED + 16, 16)] = a1 * (1.0 / T)

        pltpu.sync_copy(out_v, v_hbm.at[b])
        return carry

    lax.fori_loop(0, BPW, body, 0)


_pool = pl.kernel(
    _pool_body,
    out_type=jax.ShapeDtypeStruct((B, VPAD), jnp.float32),
    mesh=plsc.VectorSubcoreMesh(
        core_axis_name="c", subcore_axis_name="s", num_cores=NC, num_subcores=NS
    ),
    scratch_types=[
        pltpu.VMEM((NSTREAM, SLEN), jnp.int32),
        pltpu.VMEM((IDS_PAD, ED), jnp.float32),
        pltpu.VMEM((T, CONT), jnp.float32),
        pltpu.VMEM((VPAD,), jnp.float32),
        pltpu.SemaphoreType.DMA,
    ],
    compiler_params=pltpu.CompilerParams(use_tc_tiling_on_sc=False),
)

BB = 512


def _mix_body(z_ref, v_ref, wp_ref, bp_ref, wg_ref, bg_ref, o_ref):
    zb = z_ref[...]
    gate = jax.nn.sigmoid(
        jnp.dot(zb, wg_ref[...], preferred_element_type=jnp.float32) + bg_ref[...]
    )
    exo = (
        jnp.dot(v_ref[...], wp_ref[...], preferred_element_type=jnp.float32)
        + bp_ref[...]
    )
    o_ref[...] = zb + gate * exo


def _mix(z, v, wp, bp, wg, bg):
    return pl.pallas_call(
        _mix_body,
        grid=(B // BB,),
        in_specs=[
            pl.BlockSpec((BB, ZD), lambda i: (i, 0)),
            pl.BlockSpec((BB, VPAD), lambda i: (i, 0)),
            pl.BlockSpec((VPAD, ZD), lambda i: (0, 0)),
            pl.BlockSpec((1, ZD), lambda i: (0, 0)),
            pl.BlockSpec((ZD, ZD), lambda i: (0, 0)),
            pl.BlockSpec((1, ZD), lambda i: (0, 0)),
        ],
        out_specs=pl.BlockSpec((BB, ZD), lambda i: (i, 0)),
        out_shape=jax.ShapeDtypeStruct((B, ZD), jnp.float32),
    )(z, v, wp, bp, wg, bg)


def kernel(z, past_exo_cont, past_exo_cat, tables, W_proj, b_proj, W_gate, b_gate):
    ids = jnp.clip(past_exo_cat, 0, V - 1).astype(jnp.int32)
    off = jnp.arange(NCAT, dtype=jnp.int32) * V
    flat = (ids + off[None, None, :]).reshape(B, NIDS)
    flat = jnp.pad(flat, ((0, 0), (0, IDS_PAD - NIDS)))
    flat = flat.reshape(B, NSTREAM, SLEN)
    tab = tables.reshape(NCAT * V, ED)

    v = _pool(flat, past_exo_cont, tab)

    wp = jnp.concatenate(
        [W_proj, jnp.zeros((VPAD - IN_DIM, ZD), W_proj.dtype)], axis=0
    )
    return _mix(z, v, wp, b_proj.reshape(1, ZD), W_gate, b_gate.reshape(1, ZD))
